# baseline (speedup 1.0000x reference)
import functools

import jax
import jax.numpy as jnp
from jax import lax
from jax.experimental import pallas as pl
from jax.experimental.pallas import tpu as pltpu
from jax.experimental.pallas import tpu_sc as plsc

N = 10000
NP = 10240
D = 128
NC = 2
NS = 16
NW = NC * NS
C = 32
Q = 8
RB = 4
RPT = NP // NS


def _splat(wv, l):
    return lax.gather(
        wv, jnp.full((16, 1), l, jnp.int32),
        lax.GatherDimensionNumbers(
            offset_dims=(), collapsed_slice_dims=(0,), start_index_map=(0,)),
        slice_sizes=(1,),
        mode=lax.GatherScatterMode.PROMISE_IN_BOUNDS)


def _sc_scatter_fn(ch0, ch1):
    mesh = plsc.VectorSubcoreMesh(core_axis_name="c", subcore_axis_name="s")

    @functools.partial(
        pl.kernel,
        out_type=jax.ShapeDtypeStruct((NC, NP, D), jnp.float32),
        mesh=mesh,
        compiler_params=pltpu.CompilerParams(needs_layout_passes=False),
        scratch_types=[
            pltpu.VMEM((Q, C), jnp.int32),
            pltpu.VMEM((Q, C), jnp.int32),
            pltpu.VMEM((Q, C), jnp.float32),
            pltpu.VMEM((RB, C, D), jnp.float32),
            pltpu.VMEM((RB, C, D), jnp.float32),
            pltpu.VMEM_SHARED((NP, D), jnp.float32),
        ] + [pltpu.SemaphoreType.DMA] * (2 * RB + 2 * Q),
    )
    def body(src_hbm, dst_hbm, wt_hbm, sup_hbm, zeros_hbm, out_hbm,
             src_r, dst_r, wt_r, rows_v, wrows_v, acc, *sems):
        cid = lax.axis_index("c")
        sid = lax.axis_index("s")
        base = jnp.where(cid == 0, sid * ch0, NS * ch0 + sid * ch1)
        K = jnp.where(cid == 0, ch0 // Q, ch1 // Q)
        gsem = sems[:RB]
        ssem = sems[RB:2 * RB]
        wsem = sems[2 * RB:2 * RB + Q]
        dsem = sems[2 * RB + Q:]

        def sw_start(cc, q):
            g = (base + cc) * C
            pltpu.async_copy(src_hbm.at[pl.ds(g, C)], src_r.at[q], wsem[q])
            pltpu.async_copy(wt_hbm.at[pl.ds(g, C)], wt_r.at[q], wsem[q])

        def sw_wait(cc, q):
            g = (base + cc) * C
            pltpu.make_async_copy(
                src_hbm.at[pl.ds(g, C)], src_r.at[q], wsem[q]).wait()
            pltpu.make_async_copy(
                wt_hbm.at[pl.ds(g, C)], wt_r.at[q], wsem[q]).wait()

        def dst_start(cc, q):
            g = (base + cc) * C
            pltpu.async_copy(dst_hbm.at[pl.ds(g, C)], dst_r.at[q], dsem[q])

        def dst_wait(cc, q):
            g = (base + cc) * C
            pltpu.make_async_copy(
                dst_hbm.at[pl.ds(g, C)], dst_r.at[q], dsem[q]).wait()

        def gather_start(q, rb):
            pltpu.async_copy(sup_hbm.at[src_r.at[q]], rows_v.at[rb], gsem[rb])

        def gather_wait(q, rb):
            pltpu.make_async_copy(
                sup_hbm.at[src_r.at[q]], rows_v.at[rb], gsem[rb]).wait()

        def scatter_start(q, rb):
            pltpu.async_copy(
                wrows_v.at[rb], acc.at[dst_r.at[q]], ssem[rb], add=True)

        def scatter_wait(q, rb):
            pltpu.make_async_copy(
                wrows_v.at[rb], acc.at[dst_r.at[q]], ssem[rb]).wait()

        def scale(q, rb):
            for j in range(C // 16):
                wv = wt_r[q, pl.ds(16 * j, 16)]
                for l in range(16):
                    e = 16 * j + l
                    ws = _splat(wv, l)
                    for k in range(D // 16):
                        sl = pl.ds(16 * k, 16)
                        wrows_v[rb, e, sl] = rows_v[rb, e, sl] * ws

        pltpu.sync_copy(zeros_hbm, acc.at[pl.ds(sid * RPT, RPT)])
        plsc.subcore_barrier()

        for q in range(Q):
            sw_start(q, q)
        for q in range(RB):
            dst_start(q, q)
        for q in range(RB):
            sw_wait(q, q)
            gather_start(q, q)

        def octet(k, carry):
            for b in range(Q):
                cc = Q * k + b
                rb = b % RB
                qo = (b + 4) % Q
                if b < 4:
                    @pl.when(k > 0)
                    def _():
                        scatter_wait(qo, rb)
                    dst_start(cc + 4, qo)
                else:
                    scatter_wait(qo, rb)

                    @pl.when(k < K - 1)
                    def _():
                        dst_start(cc + 4, qo)
                gather_wait(b, rb)
                scale(b, rb)
                @pl.when(k < K - 1)
                def _():
                    sw_start(cc + Q, b)
                if b < 4:
                    sw_wait(cc + 4, qo)
                    gather_start(qo, rb)
                else:
                    @pl.when(k < K - 1)
                    def _():
                        sw_wait(cc + 4, qo)
                        gather_start(qo, rb)
                dst_wait(cc, b)
                scatter_start(b, rb)
            return carry

        lax.fori_loop(0, K, octet, 0, unroll=False)
        for b in range(4):
            scatter_wait(b + 4, b)
        plsc.subcore_barrier()
        pltpu.sync_copy(acc.at[pl.ds(sid * RPT, RPT)],
                        out_hbm.at[cid, pl.ds(sid * RPT, RPT)])

    return body


def _mm1_body(x_ref, w_ref, o_ref):
    o_ref[...] = jnp.dot(x_ref[...], w_ref[...],
                         preferred_element_type=jnp.float32)


def _mm2_body(p_ref, b_ref, w_ref, o_ref):
    h = (p_ref[0] + p_ref[1])[:N] + b_ref[...]
    o_ref[...] = jnp.dot(h, w_ref[...], preferred_element_type=jnp.float32)


def _fin_body(x_ref, p_ref, b_ref, o_ref):
    o_ref[...] = (x_ref[...] + (p_ref[0] + p_ref[1])[:N] + b_ref[...]) * 0.5


def kernel(x, edge_index, edge_weight, W1, b1, W2, b2):
    E = edge_weight.shape[0]
    ch = -(-E // (NW * C))
    ch = -(-ch // 8) * 8
    ep = NW * ch * C
    ch0 = int(2 * ch * 0.80 / 8 + 0.5) * 8
    ch1 = 2 * ch - ch0
    src = edge_index[0].astype(jnp.int32)
    dst = edge_index[1].astype(jnp.int32)
    wt = edge_weight.astype(jnp.float32)
    pad = ep - E
    src_p = jnp.concatenate([src, jnp.zeros((pad,), jnp.int32)])
    dst_p = jnp.concatenate([dst, jnp.zeros((pad,), jnp.int32)])
    wt_p = jnp.concatenate([wt, jnp.zeros((pad,), jnp.float32)])
    zeros = jnp.zeros((RPT, D), jnp.float32)

    mm1 = pl.pallas_call(
        _mm1_body, out_shape=jax.ShapeDtypeStruct((N, D), jnp.float32))
    mm2 = pl.pallas_call(
        _mm2_body, out_shape=jax.ShapeDtypeStruct((N, D), jnp.float32))
    fin = pl.pallas_call(
        _fin_body, out_shape=jax.ShapeDtypeStruct((N, D), jnp.float32))
    scatter = _sc_scatter_fn(ch0, ch1)

    sup1 = mm1(x, W1)
    parts1 = scatter(src_p, dst_p, wt_p, sup1, zeros)
    sup2 = mm2(parts1, b1, W2)
    parts2 = scatter(src_p, dst_p, wt_p, sup2, zeros)
    return fin(x, parts2, b2)


# speedup vs baseline: 1.0337x; 1.0337x over previous
import functools

import jax
import jax.numpy as jnp
from jax import lax
from jax.experimental import pallas as pl
from jax.experimental.pallas import tpu as pltpu
from jax.experimental.pallas import tpu_sc as plsc

N = 10000
NP = 10240
D = 128
NC = 2
NS = 16
NW = NC * NS
C = 32
Q = 8
RB = 4
RPT = NP // NS


def _splat(wv, l):
    return lax.gather(
        wv, jnp.full((16, 1), l, jnp.int32),
        lax.GatherDimensionNumbers(
            offset_dims=(), collapsed_slice_dims=(0,), start_index_map=(0,)),
        slice_sizes=(1,),
        mode=lax.GatherScatterMode.PROMISE_IN_BOUNDS)


def _sc_scatter_fn(ch0, ch1):
    mesh = plsc.VectorSubcoreMesh(core_axis_name="c", subcore_axis_name="s")

    @functools.partial(
        pl.kernel,
        out_type=jax.ShapeDtypeStruct((NC, NP, D), jnp.float32),
        mesh=mesh,
        compiler_params=pltpu.CompilerParams(needs_layout_passes=False),
        scratch_types=[
            pltpu.VMEM((Q, C), jnp.int32),
            pltpu.VMEM((Q, C), jnp.int32),
            pltpu.VMEM((Q, C), jnp.float32),
            pltpu.VMEM((RB, C, D), jnp.float32),
            pltpu.VMEM((RB, C, D), jnp.float32),
            pltpu.VMEM_SHARED((NP, D), jnp.float32),
        ] + [pltpu.SemaphoreType.DMA] * (2 * RB + 2 * Q),
    )
    def body(src_hbm, dst_hbm, wt_hbm, sup_hbm, zeros_hbm, out_hbm,
             src_r, dst_r, wt_r, rows_v, wrows_v, acc, *sems):
        cid = lax.axis_index("c")
        sid = lax.axis_index("s")
        base = jnp.where(cid == 0, sid * ch0, NS * ch0 + sid * ch1)
        K = jnp.where(cid == 0, ch0 // Q, ch1 // Q)
        gsem = sems[:RB]
        ssem = sems[RB:2 * RB]
        wsem = sems[2 * RB:2 * RB + Q]
        dsem = sems[2 * RB + Q:]

        def sw_start(cc, q):
            g = (base + cc) * C
            pltpu.async_copy(src_hbm.at[pl.ds(g, C)], src_r.at[q], wsem[q])
            pltpu.async_copy(wt_hbm.at[pl.ds(g, C)], wt_r.at[q], wsem[q])

        def sw_wait(cc, q):
            g = (base + cc) * C
            pltpu.make_async_copy(
                src_hbm.at[pl.ds(g, C)], src_r.at[q], wsem[q]).wait()
            pltpu.make_async_copy(
                wt_hbm.at[pl.ds(g, C)], wt_r.at[q], wsem[q]).wait()

        def dst_start(cc, q):
            g = (base + cc) * C
            pltpu.async_copy(dst_hbm.at[pl.ds(g, C)], dst_r.at[q], dsem[q])

        def dst_wait(cc, q):
            g = (base + cc) * C
            pltpu.make_async_copy(
                dst_hbm.at[pl.ds(g, C)], dst_r.at[q], dsem[q]).wait()

        def gather_start(q, rb):
            pltpu.async_copy(sup_hbm.at[src_r.at[q]], rows_v.at[rb], gsem[rb])

        def gather_wait(q, rb):
            pltpu.make_async_copy(
                sup_hbm.at[src_r.at[q]], rows_v.at[rb], gsem[rb]).wait()

        def scatter_start(q, rb):
            pltpu.async_copy(
                wrows_v.at[rb], acc.at[dst_r.at[q]], ssem[rb], add=True)

        def scatter_wait(q, rb):
            pltpu.make_async_copy(
                wrows_v.at[rb], acc.at[dst_r.at[q]], ssem[rb]).wait()

        def scale(q, rb):
            for j in range(C // 16):
                wv = wt_r[q, pl.ds(16 * j, 16)]
                for l in range(16):
                    e = 16 * j + l
                    ws = _splat(wv, l)
                    for k in range(D // 16):
                        sl = pl.ds(16 * k, 16)
                        wrows_v[rb, e, sl] = rows_v[rb, e, sl] * ws

        pltpu.sync_copy(zeros_hbm, acc.at[pl.ds(sid * RPT, RPT)])
        plsc.subcore_barrier()

        for q in range(Q):
            sw_start(q, q)
        for q in range(RB):
            dst_start(q, q)
        for q in range(RB):
            sw_wait(q, q)
            gather_start(q, q)

        def octet(k, carry):
            for b in range(Q):
                cc = Q * k + b
                rb = b % RB
                qo = (b + 4) % Q
                if b < 4:
                    @pl.when(k > 0)
                    def _():
                        scatter_wait(qo, rb)
                    dst_start(cc + 4, qo)
                else:
                    scatter_wait(qo, rb)

                    @pl.when(k < K - 1)
                    def _():
                        dst_start(cc + 4, qo)
                gather_wait(b, rb)
                scale(b, rb)
                @pl.when(k < K - 1)
                def _():
                    sw_start(cc + Q, b)
                if b < 4:
                    sw_wait(cc + 4, qo)
                    gather_start(qo, rb)
                else:
                    @pl.when(k < K - 1)
                    def _():
                        sw_wait(cc + 4, qo)
                        gather_start(qo, rb)
                dst_wait(cc, b)
                scatter_start(b, rb)
            return carry

        lax.fori_loop(0, K, octet, 0, unroll=False)
        for b in range(4):
            scatter_wait(b + 4, b)
        plsc.subcore_barrier()
        pltpu.sync_copy(acc.at[pl.ds(sid * RPT, RPT)],
                        out_hbm.at[cid, pl.ds(sid * RPT, RPT)])

    return body


def _mm1_body(x_ref, w_ref, o_ref):
    o_ref[...] = jnp.dot(x_ref[...], w_ref[...],
                         preferred_element_type=jnp.float32)


def _mm2_body(p_ref, b_ref, w_ref, o_ref):
    h = (p_ref[0] + p_ref[1])[:N] + b_ref[...]
    o_ref[...] = jnp.dot(h, w_ref[...], preferred_element_type=jnp.float32)


def _fin_body(x_ref, p_ref, b_ref, o_ref):
    o_ref[...] = (x_ref[...] + (p_ref[0] + p_ref[1])[:N] + b_ref[...]) * 0.5


def kernel(x, edge_index, edge_weight, W1, b1, W2, b2):
    E = edge_weight.shape[0]
    ch = -(-E // (NW * C))
    ch = -(-ch // 8) * 8
    ep = NW * ch * C
    ch0 = int(2 * ch * 0.7125 / 8 + 0.5) * 8
    ch1 = 2 * ch - ch0
    src = edge_index[0].astype(jnp.int32)
    dst = edge_index[1].astype(jnp.int32)
    wt = edge_weight.astype(jnp.float32)
    pad = ep - E
    src_p = jnp.concatenate([src, jnp.zeros((pad,), jnp.int32)])
    dst_p = jnp.concatenate([dst, jnp.zeros((pad,), jnp.int32)])
    wt_p = jnp.concatenate([wt, jnp.zeros((pad,), jnp.float32)])
    zeros = jnp.zeros((RPT, D), jnp.float32)

    mm1 = pl.pallas_call(
        _mm1_body, out_shape=jax.ShapeDtypeStruct((N, D), jnp.float32))
    mm2 = pl.pallas_call(
        _mm2_body, out_shape=jax.ShapeDtypeStruct((N, D), jnp.float32))
    fin = pl.pallas_call(
        _fin_body, out_shape=jax.ShapeDtypeStruct((N, D), jnp.float32))
    scatter = _sc_scatter_fn(ch0, ch1)

    sup1 = mm1(x, W1)
    parts1 = scatter(src_p, dst_p, wt_p, sup1, zeros)
    sup2 = mm2(parts1, b1, W2)
    parts2 = scatter(src_p, dst_p, wt_p, sup2, zeros)
    return fin(x, parts2, b2)
